# baseline (speedup 1.0000x reference)
import dataclasses
import functools

import jax
import jax.numpy as jnp
from jax import lax
from jax.experimental import pallas as pl
from jax.experimental.pallas import tpu as pltpu
from jax.experimental.pallas import tpu_sc as plsc

BATCH = 16384
EMB = 32
LANES = 16
NUM_CORES = 2
NUM_SUBCORES = 16
NW = NUM_CORES * NUM_SUBCORES
BPW = BATCH // NW
DEPTH = 12
NGROUP = BPW // LANES


def _dot_kernel(wt_hbm, ht_hbm, xt_hbm, out_hbm,
                pan, out_v, uidx_v, iidx_v, idx_sem, psem):
    wid = lax.axis_index("s") * NUM_CORES + lax.axis_index("c")
    base = wid * BPW

    cu = pltpu.async_copy(xt_hbm.at[0, pl.ds(base, BPW)], uidx_v, idx_sem)
    ci = pltpu.async_copy(xt_hbm.at[1, pl.ds(base, BPW)], iidx_v, idx_sem)
    cu.wait()
    ci.wait()

    def fire(u, i, k):
        ub = pl.multiple_of((u >> 7) * 128, 128)
        ib = pl.multiple_of((i >> 7) * 128, 128)
        pltpu.async_copy(wt_hbm.at[:, pl.ds(ub, 128)], pan.at[k, 0],
                         psem.at[k])
        pltpu.async_copy(ht_hbm.at[:, pl.ds(ib, 128)], pan.at[k, 1],
                         psem.at[k])

    def drain(k):
        pltpu.make_async_copy(wt_hbm.at[:, pl.ds(0, 128)], pan.at[k, 0],
                              psem.at[k]).wait()
        pltpu.make_async_copy(ht_hbm.at[:, pl.ds(0, 128)], pan.at[k, 1],
                              psem.at[k]).wait()

    iota16 = lax.iota(jnp.int32, LANES)
    last_lane = iota16 == (LANES - 1)

    u_first = uidx_v[pl.ds(0, LANES)]
    i_first = iidx_v[pl.ds(0, LANES)]
    for j in range(DEPTH):
        fire(u_first[j], i_first[j], j)

    @pl.loop(0, NGROUP)
    def _(g):
        s0 = g * LANES
        u_cur = uidx_v[pl.ds(s0, LANES)]
        i_cur = iidx_v[pl.ds(s0, LANES)]
        ucol = u_cur & 127
        icol = i_cur & 127
        for j in range(LANES):
            k = lax.rem(s0 + j, DEPTH)
            drain(k)
            kv = jnp.full((LANES,), k, jnp.int32)
            ucolv = jnp.full((LANES,), ucol[j], jnp.int32)
            icolv = jnp.full((LANES,), icol[j], jnp.int32)
            zero = jnp.zeros((LANES,), jnp.int32)
            one = jnp.full((LANES,), 1, jnp.int32)
            u0 = plsc.load_gather(pan, [kv, zero, iota16, ucolv])
            u1 = plsc.load_gather(pan, [kv, zero, iota16 + LANES, ucolv])
            v0 = plsc.load_gather(pan, [kv, one, iota16, icolv])
            v1 = plsc.load_gather(pan, [kv, one, iota16 + LANES, icolv])
            p = u0 * v0 + u1 * v1
            c = plsc.cumsum(p)
            plsc.store_scatter(
                out_v, [jnp.full((LANES,), s0 + j, jnp.int32)], c,
                mask=last_lane)
            if j + DEPTH < LANES:
                fire(u_cur[j + DEPTH], i_cur[j + DEPTH], k)
            else:
                @pl.when(g < NGROUP - 1)
                def _():
                    u_nxt = uidx_v[pl.ds(s0 + LANES, LANES)]
                    i_nxt = iidx_v[pl.ds(s0 + LANES, LANES)]
                    fire(u_nxt[j + DEPTH - LANES],
                         i_nxt[j + DEPTH - LANES], k)

    pltpu.sync_copy(out_v, out_hbm.at[pl.ds(base, BPW)])


@jax.jit
def _mf_dot(xt, Wt, Ht):
    mesh = plsc.VectorSubcoreMesh(core_axis_name="c", subcore_axis_name="s")
    cp = pltpu.CompilerParams()
    if "needs_layout_passes" in pltpu.CompilerParams.__dataclass_fields__:
        cp = dataclasses.replace(cp, needs_layout_passes=False)
    cp = dataclasses.replace(cp, use_tc_tiling_on_sc=True)
    grid_kernel = pl.kernel(
        _dot_kernel,
        out_type=jax.ShapeDtypeStruct((BATCH,), jnp.float32),
        mesh=mesh,
        scratch_types=[
            pltpu.VMEM((DEPTH, 2, EMB, 128), jnp.float32),
            pltpu.VMEM((BPW,), jnp.float32),
            pltpu.VMEM((BPW,), jnp.int32),
            pltpu.VMEM((BPW,), jnp.int32),
            pltpu.SemaphoreType.DMA,
            pltpu.SemaphoreType.DMA((DEPTH,)),
        ],
        compiler_params=cp,
    )
    return grid_kernel(Wt, Ht, xt)


def kernel(x, W, H):
    return _mf_dot(x.T.astype(jnp.int32), W.T, H.T)
